# baseline (speedup 1.0000x reference)
import functools
import jax
import jax.numpy as jnp
from jax import lax
from jax.experimental import pallas as pl
from jax.experimental.pallas import tpu as pltpu
from jax.experimental.pallas import tpu_sc as plsc

_Q = 1000
_K = 4
_B = 1024
_S = 20
_NW = 16
_BPW = _B // _NW


def _sc_body(qm1_hbm, r_hbm, out_hbm, qv, rv, buf, sem):
    wid = lax.axis_index("s") * 1 + lax.axis_index("c")
    base_row = wid * (_BPW * _S)

    pltpu.sync_copy(qm1_hbm.at[pl.ds(base_row, _BPW * _S)],
                    qv.at[pl.ds(0, _BPW * _S)])
    pltpu.sync_copy(r_hbm.at[pl.ds(base_row, _BPW * _S)],
                    rv.at[pl.ds(0, _BPW * _S)])

    iota = lax.iota(jnp.int32, 16)
    zero_i = jnp.zeros((16,), jnp.int32)
    zero_f = jnp.zeros((16,), jnp.float32)
    group_rows = (iota, iota + 16)
    group_ok = (iota >= 0, iota + 16 < _S)

    for s in range(_S):
        def zrow(i, carry, s=s):
            for u in range(8):
                buf[0, s, pl.ds(i * 128 + u * 16, 16)] = zero_f
            return carry
        lax.fori_loop(0, _Q * _K // 128, zrow, 0)

    def dma(b_local):
        return pltpu.make_async_copy(
            buf, out_hbm.at[pl.ds(wid * _BPW + b_local, 1), :, :], sem)

    def scatter(b_local, reset):
        for g in range(2):
            rows = group_rows[g]
            off = b_local * _S + g * 16
            q16 = plsc.load_gather(qv, [iota + off])
            valid = group_ok[g] & (q16 >= 0)
            if reset:
                rf3 = None
            else:
                r16 = plsc.load_gather(rv, [iota + off])
                rf3 = r16.astype(jnp.float32) * (1.0 / (_K - 1))
            for k in range(_K):
                if reset:
                    w = zero_f
                else:
                    w = 1.0 - jnp.abs(rf3 - (k / (_K - 1)))
                plsc.store_scatter(buf, [zero_i, rows, q16 + (k * _Q)],
                                   w, mask=valid)

    for b_local in range(_BPW):
        if b_local >= 1:
            dma(b_local - 1).wait()
            scatter(b_local - 1, True)
        scatter(b_local, False)
        dma(b_local).start()

    dma(_BPW - 1).wait()


def kernel(question_ids, responses):
    B, S = responses.shape
    qm1 = question_ids.astype(jnp.int32).reshape(B * S) - 1
    r = responses.astype(jnp.int32).reshape(B * S)
    mesh = plsc.VectorSubcoreMesh(core_axis_name="c", subcore_axis_name="s", num_cores=1)
    f = functools.partial(
        pl.kernel,
        out_type=jax.ShapeDtypeStruct((B, S, _K * _Q), jnp.float32),
        mesh=mesh,
        compiler_params=pltpu.CompilerParams(needs_layout_passes=False),
        scratch_types=[
            pltpu.VMEM((_BPW * _S + 16,), jnp.int32),
            pltpu.VMEM((_BPW * _S + 16,), jnp.int32),
            pltpu.VMEM((1, _S, _K * _Q), jnp.float32),
            pltpu.SemaphoreType.DMA,
        ],
    )(_sc_body)
    return f(qm1, r)


# speedup vs baseline: 1.2364x; 1.2364x over previous
import functools
import jax
import jax.numpy as jnp
from jax import lax
from jax.experimental import pallas as pl
from jax.experimental.pallas import tpu as pltpu
from jax.experimental.pallas import tpu_sc as plsc

_Q = 1000
_K = 4
_B = 1024
_S = 20
_NW = 32
_BPW = _B // _NW


def _sc_body(qm1_hbm, r_hbm, out_hbm, qv, rv, buf, sem):
    wid = lax.axis_index("s") * 2 + lax.axis_index("c")
    base_row = wid * (_BPW * _S)

    pltpu.sync_copy(qm1_hbm.at[pl.ds(base_row, _BPW * _S)],
                    qv.at[pl.ds(0, _BPW * _S)])
    pltpu.sync_copy(r_hbm.at[pl.ds(base_row, _BPW * _S)],
                    rv.at[pl.ds(0, _BPW * _S)])

    iota = lax.iota(jnp.int32, 16)
    zero_i = jnp.zeros((16,), jnp.int32)
    zero_f = jnp.zeros((16,), jnp.float32)
    group_rows = (iota, iota + 16)
    group_ok = (iota >= 0, iota + 16 < _S)

    for s in range(_S):
        def zrow(i, carry, s=s):
            for u in range(8):
                buf[0, s, pl.ds(i * 128 + u * 16, 16)] = zero_f
            return carry
        lax.fori_loop(0, _Q * _K // 128, zrow, 0)

    def dma(b_local):
        return pltpu.make_async_copy(
            buf, out_hbm.at[pl.ds(wid * _BPW + b_local, 1), :, :], sem)

    def scatter(b_local, reset):
        for g in range(2):
            rows = group_rows[g]
            off = b_local * _S + g * 16
            q16 = plsc.load_gather(qv, [iota + off])
            valid = group_ok[g] & (q16 >= 0)
            if reset:
                rf3 = None
            else:
                r16 = plsc.load_gather(rv, [iota + off])
                rf3 = r16.astype(jnp.float32) * (1.0 / (_K - 1))
            for k in range(_K):
                if reset:
                    w = zero_f
                else:
                    w = 1.0 - jnp.abs(rf3 - (k / (_K - 1)))
                plsc.store_scatter(buf, [zero_i, rows, q16 + (k * _Q)],
                                   w, mask=valid)

    for b_local in range(_BPW):
        if b_local >= 1:
            dma(b_local - 1).wait()
            scatter(b_local - 1, True)
        scatter(b_local, False)
        dma(b_local).start()

    dma(_BPW - 1).wait()


def kernel(question_ids, responses):
    B, S = responses.shape
    qm1 = question_ids.astype(jnp.int32).reshape(B * S) - 1
    r = responses.astype(jnp.int32).reshape(B * S)
    mesh = plsc.VectorSubcoreMesh(core_axis_name="c", subcore_axis_name="s")
    f = functools.partial(
        pl.kernel,
        out_type=jax.ShapeDtypeStruct((B, S, _K * _Q), jnp.float32),
        mesh=mesh,
        compiler_params=pltpu.CompilerParams(needs_layout_passes=False),
        scratch_types=[
            pltpu.VMEM((_BPW * _S + 16,), jnp.int32),
            pltpu.VMEM((_BPW * _S + 16,), jnp.int32),
            pltpu.VMEM((1, _S, _K * _Q), jnp.float32),
            pltpu.SemaphoreType.DMA,
        ],
    )(_sc_body)
    return f(qm1, r)
